# baseline (speedup 1.0000x reference)
import dataclasses

import jax
import jax.numpy as jnp
from jax import lax
from jax.experimental import pallas as pl
from jax.experimental.pallas import tpu as pltpu
from jax.experimental.pallas import tpu_sc as plsc

_NC, _NS = 2, 16
_NW = _NC * _NS
_R = 4


def kernel(indices, embed_table):
    batch, seq = indices.shape
    vocab, embed_dim = embed_table.shape
    pack = 128 // embed_dim
    table128 = embed_table.reshape(vocab // pack, 128)
    idx = indices.reshape(batch * seq).astype(jnp.int32)

    bpw = batch // _NW
    nch = -(-seq // 16)
    spad = nch * 16
    s0 = (seq // 128) * 128
    s1 = seq - s0
    ostep = seq * embed_dim

    mesh = plsc.VectorSubcoreMesh(core_axis_name="c", subcore_axis_name="s")
    cp = pltpu.CompilerParams()
    if "needs_layout_passes" in pltpu.CompilerParams.__dataclass_fields__:
        cp = dataclasses.replace(cp, needs_layout_passes=False)

    @pl.kernel(
        out_type=jax.ShapeDtypeStruct((batch * seq * embed_dim,),
                                      embed_table.dtype),
        mesh=mesh,
        compiler_params=cp,
        scratch_types=(
            [pltpu.VMEM((spad,), jnp.int32)] * _R
            + [pltpu.VMEM((spad,), jnp.int32)] * _R
            + [pltpu.VMEM((spad,), jnp.int32)] * _R
            + [pltpu.VMEM((spad, 128), jnp.float32)] * _R
            + [pltpu.VMEM((spad * embed_dim,), jnp.float32)] * 2
            + [pltpu.SemaphoreType.DMA] * (2 * _R + 2)
        ),
    )
    def gather_kernel(table_hbm, idx_hbm, out_hbm, *sc):
        idx_v = sc[0:_R]
        hi_v = sc[_R:2 * _R]
        off_v = sc[2 * _R:3 * _R]
        rows_v = sc[3 * _R:4 * _R]
        opk_v = sc[4 * _R:4 * _R + 2]
        isem = sc[4 * _R + 2:5 * _R + 2]
        gsem = sc[5 * _R + 2:6 * _R + 2]
        wsem = sc[6 * _R + 2:6 * _R + 4]

        wid = lax.axis_index("s") * _NC + lax.axis_index("c")
        b0 = wid * bpw
        iota16 = lax.iota(jnp.int32, 16)

        def fire_idx(s, j):
            pltpu.async_copy(idx_hbm.at[pl.ds((b0 + s) * seq, seq)],
                             idx_v[j].at[pl.ds(0, seq)], isem[j])

        def wait_idx(j):
            pltpu.make_async_copy(idx_hbm.at[pl.ds(0, seq)],
                                  idx_v[j].at[pl.ds(0, seq)],
                                  isem[j]).wait()

        def comp(j):
            for k in range(nch):
                sl = pl.ds(k * 16, 16)
                v = idx_v[j][sl]
                hi_v[j][sl] = jax.lax.shift_right_logical(v, 2)
                off_v[j][sl] = (v & (pack - 1)) * embed_dim

        def fire_gather(j):
            pltpu.async_copy(table_hbm.at[hi_v[j].at[pl.ds(0, s0)]],
                             rows_v[j].at[pl.ds(0, s0)], gsem[j])
            pltpu.async_copy(table_hbm.at[hi_v[j].at[pl.ds(s0, s1)]],
                             rows_v[j].at[pl.ds(s0, s1)], gsem[j])

        def wait_gather(j):
            pltpu.make_async_copy(table_hbm.at[hi_v[j].at[pl.ds(0, s0)]],
                                  rows_v[j].at[pl.ds(0, s0)],
                                  gsem[j]).wait()
            pltpu.make_async_copy(table_hbm.at[hi_v[j].at[pl.ds(s0, s1)]],
                                  rows_v[j].at[pl.ds(s0, s1)],
                                  gsem[j]).wait()

        def extract(j, jo):
            @pl.loop(0, nch)
            def _(c):
                rows16 = iota16 + c * 16
                offv = off_v[j][pl.ds(c * 16, 16)]
                qbase = rows16 * embed_dim
                for l in range(embed_dim):
                    vals = plsc.load_gather(rows_v[j], [rows16, offv + l])
                    plsc.store_scatter(opk_v[jo], [qbase + l], vals)

        def fire_wb(s, jo):
            pltpu.async_copy(opk_v[jo].at[pl.ds(0, ostep)],
                             out_hbm.at[pl.ds((b0 + s) * ostep, ostep)],
                             wsem[jo])

        def wait_wb(jo):
            pltpu.make_async_copy(opk_v[jo].at[pl.ds(0, ostep)],
                                  out_hbm.at[pl.ds(0, ostep)],
                                  wsem[jo]).wait()

        fire_idx(0, 0)
        fire_idx(1, 1)
        fire_idx(2, 2)
        wait_idx(0)
        comp(0)
        fire_gather(0)
        wait_idx(1)
        comp(1)
        fire_gather(1)

        @pl.loop(0, bpw // _R)
        def _(k):
            for j in range(_R):
                s = k * _R + j
                jo = j % 2

                @pl.when(s + 3 < bpw)
                def _():
                    fire_idx(s + 3, (j + 3) % _R)

                @pl.when(s + 2 < bpw)
                def _():
                    wait_idx((j + 2) % _R)
                    comp((j + 2) % _R)
                    fire_gather((j + 2) % _R)

                wait_gather(j)

                @pl.when(s >= 2)
                def _():
                    wait_wb(jo)

                extract(j, jo)
                fire_wb(s, jo)

        wait_wb(0)
        wait_wb(1)

    out = gather_kernel(table128, idx)
    return out.reshape(batch, seq, embed_dim)


# speedup vs baseline: 1.0881x; 1.0881x over previous
import dataclasses

import jax
import jax.numpy as jnp
from jax import lax
from jax.experimental import pallas as pl
from jax.experimental.pallas import tpu as pltpu
from jax.experimental.pallas import tpu_sc as plsc

_NC, _NS = 2, 16
_NW = _NC * _NS


def kernel(indices, embed_table):
    batch, seq = indices.shape
    vocab, embed_dim = embed_table.shape
    pack = 128 // embed_dim
    table128 = embed_table.reshape(vocab // pack, 128)
    idx = indices.reshape(batch * seq).astype(jnp.int32)

    bpw = batch // _NW
    nst = bpw
    nch = -(-seq // 16)
    spad = nch * 16
    splits = [(o, min(128, seq - o)) for o in range(0, seq, 128)]

    mesh = plsc.VectorSubcoreMesh(core_axis_name="c", subcore_axis_name="s")
    cp = pltpu.CompilerParams()
    if "needs_layout_passes" in pltpu.CompilerParams.__dataclass_fields__:
        cp = dataclasses.replace(cp, needs_layout_passes=False)

    @pl.kernel(
        out_type=jax.ShapeDtypeStruct((batch, seq, embed_dim),
                                      embed_table.dtype),
        mesh=mesh,
        compiler_params=cp,
        scratch_types=(
            [pltpu.VMEM((spad,), jnp.int32)] * 2
            + [pltpu.VMEM((spad,), jnp.int32)] * 2
            + [pltpu.VMEM((spad,), jnp.int32)] * 2
            + [pltpu.VMEM((spad, 128), jnp.float32)] * 2
            + [pltpu.VMEM((spad, embed_dim), jnp.float32)] * 2
            + [pltpu.SemaphoreType.DMA] * 6
        ),
    )
    def gather_kernel(table_hbm, idx_hbm, out_hbm,
                      idx_v0, idx_v1, hi_v0, hi_v1, off_v0, off_v1,
                      rows_v0, rows_v1, out_v0, out_v1,
                      isem0, isem1, gsem0, gsem1, wsem0, wsem1):
        idx_v = (idx_v0, idx_v1)
        hi_v = (hi_v0, hi_v1)
        off_v = (off_v0, off_v1)
        rows_v = (rows_v0, rows_v1)
        out_v = (out_v0, out_v1)
        isem = (isem0, isem1)
        gsem = (gsem0, gsem1)
        wsem = (wsem0, wsem1)
        wid = lax.axis_index("s") * _NC + lax.axis_index("c")
        b0 = wid * bpw
        iota16 = lax.iota(jnp.int32, 16)

        def fire_idx(s, j):
            pltpu.async_copy(idx_hbm.at[pl.ds((b0 + s) * seq, seq)],
                             idx_v[j].at[pl.ds(0, seq)], isem[j])

        def wait_idx(j):
            pltpu.make_async_copy(idx_hbm.at[pl.ds(0, seq)],
                                  idx_v[j].at[pl.ds(0, seq)],
                                  isem[j]).wait()

        def comp(j):
            for k in range(nch):
                sl = pl.ds(k * 16, 16)
                v = idx_v[j][sl]
                hi_v[j][sl] = jax.lax.shift_right_logical(v, 2)
                off_v[j][sl] = (v & (pack - 1)) * embed_dim

        def fire_gather(j):
            for o, n in splits:
                pltpu.async_copy(table_hbm.at[hi_v[j].at[pl.ds(o, n)]],
                                 rows_v[j].at[pl.ds(o, n)], gsem[j])

        def wait_gather(j):
            pltpu.make_async_copy(table_hbm.at[pl.ds(0, seq)],
                                  rows_v[j].at[pl.ds(0, seq)],
                                  gsem[j]).wait()

        def extract(j):
            @pl.loop(0, nch)
            def _(c):
                rows16 = iota16 + c * 16
                offv = off_v[j][pl.ds(c * 16, 16)]
                for l in range(embed_dim):
                    vals = plsc.load_gather(rows_v[j], [rows16, offv + l])
                    plsc.store_scatter(out_v[j],
                                       [rows16, jnp.full((16,), l, jnp.int32)],
                                       vals)

        def fire_wb(s, j):
            pltpu.async_copy(out_v[j].at[pl.ds(0, seq)],
                             out_hbm.at[b0 + s], wsem[j])

        def wait_wb(j):
            pltpu.make_async_copy(out_v[j].at[pl.ds(0, seq)],
                                  out_hbm.at[0], wsem[j]).wait()

        fire_idx(0, 0)
        fire_idx(1, 1)
        wait_idx(0)
        comp(0)
        fire_gather(0)

        wait_idx(1)
        comp(1)
        fire_gather(1)
        fire_idx(2, 0)
        wait_gather(0)
        extract(0)
        fire_wb(0, 0)
        wait_idx(0)
        comp(0)
        fire_gather(0)
        fire_idx(3, 1)
        wait_gather(1)
        extract(1)
        fire_wb(1, 1)

        @pl.loop(1, nst // 2 - 1)
        def _(k):
            s = k * 2
            wait_idx(1)
            comp(1)
            fire_gather(1)
            fire_idx(s + 2, 0)
            wait_gather(0)
            wait_wb(0)
            extract(0)
            fire_wb(s, 0)
            wait_idx(0)
            comp(0)
            fire_gather(0)
            fire_idx(s + 3, 1)
            wait_gather(1)
            wait_wb(1)
            extract(1)
            fire_wb(s + 1, 1)

        wait_idx(1)
        comp(1)
        fire_gather(1)
        wait_gather(0)
        wait_wb(0)
        extract(0)
        fire_wb(nst - 2, 0)
        wait_gather(1)
        wait_wb(1)
        extract(1)
        fire_wb(nst - 1, 1)
        wait_wb(0)
        wait_wb(1)

    out = gather_kernel(table128, idx)
    return out
